# Initial kernel scaffold; baseline (speedup 1.0000x reference)
import functools

import jax
import jax.numpy as jnp
from jax import lax
from jax.experimental import pallas as pl
from jax.experimental.pallas import tpu as pltpu
from jax.experimental.pallas import tpu_sc as plsc

N = 10000
E = 320000
D = 128
P = 3
C = 32
GW = 16

NC = 2
NS = 16
NW = NC * NS
EPW = E // NW
K = 80
NCHUNK = EPW // K
RPT = N // NS


def _sc_body(ei_hbm, ej_hbm, pos_hbm, x_hbm, zs_hbm, zg_hbm,
             s_out, g_out,
             pos_v, ei_v, ej_v, xrows, grows, s_sh, g_sh, sem):
    c = lax.axis_index("c")
    s = lax.axis_index("s")
    w = s * NC + c
    lanes = lax.iota(jnp.int32, 16)

    pltpu.sync_copy(pos_hbm, pos_v)

    r0 = s * RPT
    pltpu.sync_copy(zs_hbm.at[pl.ds(r0, RPT)], s_sh.at[pl.ds(r0, RPT)])
    pltpu.sync_copy(zg_hbm.at[pl.ds(r0, RPT)], g_sh.at[pl.ds(r0, RPT)])

    pltpu.sync_copy(zg_hbm.at[pl.ds(0, K)], grows)
    ones = jnp.full((16,), 1.0, jnp.float32)
    col0 = jnp.zeros((16,), jnp.int32)
    for t in range(K // 16):
        plsc.store_scatter(grows, [t * 16 + lanes, col0], ones)

    plsc.subcore_barrier()

    ebase = w * EPW

    def chunk(ci, carry):
        base = ebase + ci * K
        pltpu.sync_copy(ei_hbm.at[pl.ds(base, K)], ei_v)
        pltpu.sync_copy(ej_hbm.at[pl.ds(base, K)], ej_v)
        pltpu.async_copy(x_hbm.at[ej_v], xrows, sem).wait()

        def geo(t, inner):
            eiv = ei_v[pl.ds(t * 16, 16)]
            ejv = ej_v[pl.ds(t * 16, 16)]
            bi = eiv * 3
            bj = ejv * 3
            pix = plsc.load_gather(pos_v, [bi])
            piy = plsc.load_gather(pos_v, [bi + 1])
            piz = plsc.load_gather(pos_v, [bi + 2])
            pjx = plsc.load_gather(pos_v, [bj])
            pjy = plsc.load_gather(pos_v, [bj + 1])
            pjz = plsc.load_gather(pos_v, [bj + 2])
            dx = pjx - pix
            dy = pjy - piy
            dz = pjz - piz
            d2 = dx * dx + dy * dy + dz * dz
            yi = jnp.int32(0x5F3759DF) - lax.shift_right_logical(
                plsc.bitcast(d2, jnp.int32), 1)
            y = plsc.bitcast(yi, jnp.float32)
            h = d2 * 0.5
            for _ in range(4):
                y = y * (1.5 - h * y * y)
            rows = t * 16 + lanes
            plsc.store_scatter(grows, [rows, col0 + 1], d2 * y)
            plsc.store_scatter(grows, [rows, col0 + 2], dx * y)
            plsc.store_scatter(grows, [rows, col0 + 3], dy * y)
            plsc.store_scatter(grows, [rows, col0 + 4], dz * y)
            return inner

        lax.fori_loop(0, K // 16, geo, 0)

        pltpu.sync_copy(xrows, s_sh.at[ei_v], add=True)
        pltpu.sync_copy(grows, g_sh.at[ei_v], add=True)
        return carry

    lax.fori_loop(0, NCHUNK, chunk, 0)

    plsc.subcore_barrier()

    pltpu.sync_copy(s_sh.at[pl.ds(r0, RPT)], s_out.at[c, pl.ds(r0, RPT)])
    pltpu.sync_copy(g_sh.at[pl.ds(r0, RPT)], g_out.at[c, pl.ds(r0, RPT)])


def _sc_segment_sums(ei, ej, pos_flat, x, zs, zg):
    mesh = plsc.VectorSubcoreMesh(core_axis_name="c", subcore_axis_name="s")
    return pl.kernel(
        _sc_body,
        out_type=(
            jax.ShapeDtypeStruct((NC, N, D), jnp.float32),
            jax.ShapeDtypeStruct((NC, N, GW), jnp.float32),
        ),
        mesh=mesh,
        scratch_types=[
            pltpu.VMEM((N * P,), jnp.float32),
            pltpu.VMEM((K,), jnp.int32),
            pltpu.VMEM((K,), jnp.int32),
            pltpu.VMEM((K, D), jnp.float32),
            pltpu.VMEM((K, GW), jnp.float32),
            pltpu.VMEM_SHARED((N, D), jnp.float32),
            pltpu.VMEM_SHARED((N, GW), jnp.float32),
            pltpu.SemaphoreType.DMA,
        ],
    )(ei, ej, pos_flat, x, zs, zg)


def _tc_body(s2_ref, g2_ref, x_ref, w1p_ref, b1_ref, a1_ref, ba1_ref,
             w2a_ref, w2b_ref, w2cp_ref, w2d_ref, b2_ref,
             a2a_ref, a2b_ref, ba2_ref, out_ref):
    f32 = jnp.float32
    g = g2_ref[0] + g2_ref[1]
    ssum = s2_ref[0] + s2_ref[1]
    cnt = g[:, :1]
    inv = 1.0 / jnp.maximum(cnt, 1.0)
    has = jnp.minimum(cnt, 1.0)
    aggr1 = jnp.dot(g, w1p_ref[...], preferred_element_type=f32) * inv \
        + has * b1_ref[...]
    ctx = jnp.dot(aggr1, a1_ref[...], preferred_element_type=f32) + ba1_ref[...]
    wx = w2a_ref[...] - w2b_ref[...]
    t = jnp.dot(x_ref[...], wx, preferred_element_type=f32) \
        + jnp.dot(ctx, w2d_ref[...], preferred_element_type=f32) + b2_ref[...]
    aggr2 = (cnt * t
             + jnp.dot(ssum, w2b_ref[...], preferred_element_type=f32)
             + jnp.dot(g, w2cp_ref[...], preferred_element_type=f32)) * inv
    out_ref[...] = jnp.dot(aggr2, a2a_ref[...], preferred_element_type=f32) \
        + jnp.dot(ctx, a2b_ref[...], preferred_element_type=f32) + ba2_ref[...]


def _tc_combine(s2, g2, x, w1p, b1, a1, ba1, w2a, w2b, w2cp, w2d, b2,
                a2a, a2b, ba2):
    bn = 1000
    grid = (N // bn,)
    full = lambda shape: pl.BlockSpec(shape, lambda i, _s=None: (0,) * len(shape))
    return pl.pallas_call(
        _tc_body,
        grid=grid,
        in_specs=[
            pl.BlockSpec((NC, bn, D), lambda i: (0, i, 0)),
            pl.BlockSpec((NC, bn, GW), lambda i: (0, i, 0)),
            pl.BlockSpec((bn, D), lambda i: (i, 0)),
            full((GW, C)), full((1, C)), full((C, C)), full((1, C)),
            full((D, D)), full((D, D)), full((GW, D)), full((C, D)),
            full((1, D)), full((D, D)), full((C, D)), full((1, D)),
        ],
        out_specs=pl.BlockSpec((bn, D), lambda i: (i, 0)),
        out_shape=jax.ShapeDtypeStruct((N, D), jnp.float32),
    )(s2, g2, x, w1p, b1, a1, ba1, w2a, w2b, w2cp, w2d, b2, a2a, a2b, ba2)


def kernel(x, edge_index, pos, W1, b1, A1, bA1, W2, b2, A2, bA2):
    ei = edge_index[0]
    ej = edge_index[1]
    pos_flat = pos.reshape(-1)
    zs = jnp.zeros((N, D), jnp.float32)
    zg = jnp.zeros((N, GW), jnp.float32)

    s2, g2 = _sc_segment_sums(ei, ej, pos_flat, x, zs, zg)

    w1p = jnp.zeros((GW, C), jnp.float32).at[1:1 + P + 1].set(W1)
    w2a = W2[:D]
    w2b = W2[D:2 * D]
    w2cp = jnp.zeros((GW, D), jnp.float32).at[1:1 + P + 1].set(W2[2 * D:2 * D + 4])
    w2d = W2[2 * D + 4:]
    a2a = A2[:D]
    a2b = A2[D:]

    return _tc_combine(
        s2, g2, x,
        w1p, b1.reshape(1, C), A1, bA1.reshape(1, C),
        w2a, w2b, w2cp, w2d, b2.reshape(1, D),
        a2a, a2b, bA2.reshape(1, D))


# speedup vs baseline: 6.9527x; 6.9527x over previous
import jax
import jax.numpy as jnp
from jax import lax
from jax.experimental import pallas as pl
from jax.experimental.pallas import tpu as pltpu
from jax.experimental.pallas import tpu_sc as plsc

N = 10000
E = 320000
D = 128
NP = 10240
P = 3
C = 32

XS = 80
X1 = D - XS
W0 = XS
W1R = 64
GB = X1

NS = 16
EPT = E // NS
K = 80
NCHUNK = EPT // K
RPT = NP // NS


def _sc_body(ei_hbm, ej_hbm, t0_hbm, t1_hbm, pp_hbm, z0_hbm, z1_hbm,
             s0_out, s1_out,
             ei_v, ej_v, r0buf, r1buf, pbi, s0_sh, s1_sh, sem):
    c = lax.axis_index("c")
    s = lax.axis_index("s")
    lanes = lax.iota(jnp.int32, 16)
    r0 = s * RPT

    @pl.when(c == 0)
    def _():
        pltpu.sync_copy(z0_hbm.at[pl.ds(r0, RPT)], s0_sh.at[pl.ds(r0, RPT)])

    @pl.when(c == 1)
    def _():
        pltpu.sync_copy(z1_hbm.at[pl.ds(r0, RPT)], s1_sh.at[pl.ds(r0, RPT)])

    plsc.subcore_barrier()

    ebase = s * EPT

    def chunk(ci, carry):
        base = pl.multiple_of(ebase + ci * K, 8)
        pltpu.sync_copy(ei_hbm.at[pl.ds(base, K)], ei_v)
        pltpu.sync_copy(ej_hbm.at[pl.ds(base, K)], ej_v)

        @pl.when(c == 0)
        def _():
            pltpu.async_copy(t0_hbm.at[ej_v], r0buf, sem).wait()
            pltpu.sync_copy(r0buf, s0_sh.at[ei_v], add=True)

        @pl.when(c == 1)
        def _():
            pltpu.async_copy(t1_hbm.at[ej_v], r1buf, sem).wait()
            pltpu.async_copy(pp_hbm.at[ei_v], pbi, sem).wait()
            for t in range(K // 16):
                rows = t * 16 + lanes
                col0 = jnp.zeros((16,), jnp.int32)
                pix = plsc.load_gather(pbi, [rows, col0])
                piy = plsc.load_gather(pbi, [rows, col0 + 1])
                piz = plsc.load_gather(pbi, [rows, col0 + 2])
                pjx = plsc.load_gather(r1buf, [rows, col0 + GB])
                pjy = plsc.load_gather(r1buf, [rows, col0 + GB + 1])
                pjz = plsc.load_gather(r1buf, [rows, col0 + GB + 2])
                dx = pjx - pix
                dy = pjy - piy
                dz = pjz - piz
                d2 = dx * dx + dy * dy + dz * dz
                yi = jnp.int32(0x5F3759DF) - lax.shift_right_logical(
                    plsc.bitcast(d2, jnp.int32), 1)
                y = plsc.bitcast(yi, jnp.float32)
                h = d2 * 0.5
                for _ in range(4):
                    y = y * (1.5 - h * y * y)
                one = jnp.full((16,), 1.0, jnp.float32)
                plsc.store_scatter(r1buf, [rows, col0 + GB], one)
                plsc.store_scatter(r1buf, [rows, col0 + GB + 1], d2 * y)
                plsc.store_scatter(r1buf, [rows, col0 + GB + 2], dx * y)
                plsc.store_scatter(r1buf, [rows, col0 + GB + 3], dy * y)
                plsc.store_scatter(r1buf, [rows, col0 + GB + 4], dz * y)
            pltpu.sync_copy(r1buf, s1_sh.at[ei_v], add=True)

        return carry

    lax.fori_loop(0, NCHUNK, chunk, 0)

    plsc.subcore_barrier()

    @pl.when(c == 0)
    def _():
        pltpu.sync_copy(s0_sh.at[pl.ds(r0, RPT)], s0_out.at[pl.ds(r0, RPT)])

    @pl.when(c == 1)
    def _():
        pltpu.sync_copy(s1_sh.at[pl.ds(r0, RPT)], s1_out.at[pl.ds(r0, RPT)])


def _sc_segment_sums(ei, ej, t0, t1, pp, z0, z1):
    mesh = plsc.VectorSubcoreMesh(core_axis_name="c", subcore_axis_name="s")
    return pl.kernel(
        _sc_body,
        out_type=(
            jax.ShapeDtypeStruct((NP, W0), jnp.float32),
            jax.ShapeDtypeStruct((NP, W1R), jnp.float32),
        ),
        mesh=mesh,
        compiler_params=pltpu.CompilerParams(
            needs_layout_passes=False, use_tc_tiling_on_sc=False),
        scratch_types=[
            pltpu.VMEM((K,), jnp.int32),
            pltpu.VMEM((K,), jnp.int32),
            pltpu.VMEM((K, W0), jnp.float32),
            pltpu.VMEM((K, W1R), jnp.float32),
            pltpu.VMEM((K, 16), jnp.float32),
            pltpu.VMEM_SHARED((NP, W0), jnp.float32),
            pltpu.VMEM_SHARED((NP, W1R), jnp.float32),
            pltpu.SemaphoreType.DMA,
        ],
    )(ei, ej, t0, t1, pp, z0, z1)


def _tc_body(s0_ref, s1_ref, x_ref, w1p_ref, b1_ref, a1_ref, ba1_ref,
             w2a_ref, w2b0_ref, w2b1_ref, w2cp_ref, w2d_ref, b2_ref,
             a2a_ref, a2b_ref, ba2_ref, out_ref):
    f32 = jnp.float32
    s0 = s0_ref[...]
    s1 = s1_ref[...]
    g = s1[:, GB:GB + 16]
    cnt = g[:, :1]
    inv = 1.0 / jnp.maximum(cnt, 1.0)
    has = jnp.minimum(cnt, 1.0)
    aggr1 = jnp.dot(g, w1p_ref[...], preferred_element_type=f32) * inv \
        + has * b1_ref[...]
    ctx = jnp.dot(aggr1, a1_ref[...], preferred_element_type=f32) + ba1_ref[...]
    wx = w2a_ref[...]
    t = jnp.dot(x_ref[...], wx, preferred_element_type=f32) \
        + jnp.dot(ctx, w2d_ref[...], preferred_element_type=f32) + b2_ref[...]
    sterm = jnp.dot(s0, w2b0_ref[...], preferred_element_type=f32) \
        + jnp.dot(s1[:, :X1], w2b1_ref[...], preferred_element_type=f32)
    aggr2 = (cnt * t + sterm
             + jnp.dot(g, w2cp_ref[...], preferred_element_type=f32)) * inv
    out_ref[...] = jnp.dot(aggr2, a2a_ref[...], preferred_element_type=f32) \
        + jnp.dot(ctx, a2b_ref[...], preferred_element_type=f32) + ba2_ref[...]


def _tc_combine(s0, s1, x, w1p, b1, a1, ba1, wx, w2b0, w2b1, w2cp, w2d, b2,
                a2a, a2b, ba2):
    bn = 1000
    grid = (N // bn,)
    full = lambda shape: pl.BlockSpec(shape, lambda i: (0,) * len(shape))
    return pl.pallas_call(
        _tc_body,
        grid=grid,
        in_specs=[
            pl.BlockSpec((bn, W0), lambda i: (i, 0)),
            pl.BlockSpec((bn, W1R), lambda i: (i, 0)),
            pl.BlockSpec((bn, D), lambda i: (i, 0)),
            full((16, C)), full((1, C)), full((C, C)), full((1, C)),
            full((D, D)), full((W0, D)), full((X1, D)), full((16, D)),
            full((C, D)), full((1, D)), full((D, D)), full((C, D)),
            full((1, D)),
        ],
        out_specs=pl.BlockSpec((bn, D), lambda i: (i, 0)),
        out_shape=jax.ShapeDtypeStruct((N, D), jnp.float32),
    )(s0, s1, x, w1p, b1, a1, ba1, wx, w2b0, w2b1, w2cp, w2d, b2,
      a2a, a2b, ba2)


def kernel(x, edge_index, pos, W1, b1, A1, bA1, W2, b2, A2, bA2):
    ei = edge_index[0]
    ej = edge_index[1]
    zpad = jnp.zeros((N, 16 - P), jnp.float32)
    t0 = x[:, :XS]
    t1 = jnp.concatenate([x[:, XS:], pos, zpad[:, :W1R - X1 - P]], axis=1)
    pp = jnp.concatenate([pos, zpad], axis=1)
    z0 = jnp.zeros((NP, W0), jnp.float32)
    z1 = jnp.zeros((NP, W1R), jnp.float32)

    s0, s1 = _sc_segment_sums(ei, ej, t0, t1, pp, z0, z1)

    w1p = jnp.zeros((16, C), jnp.float32).at[1:1 + P + 1].set(W1)
    wx = W2[:D] - W2[D:2 * D]
    w2b0 = W2[D:D + XS]
    w2b1 = W2[D + XS:2 * D]
    w2cp = jnp.zeros((16, D), jnp.float32).at[1:1 + P + 1].set(
        W2[2 * D:2 * D + P + 1])
    w2d = W2[2 * D + P + 1:]
    a2a = A2[:D]
    a2b = A2[D:]

    return _tc_combine(
        s0, s1, x,
        w1p, b1.reshape(1, C), A1, bA1.reshape(1, C),
        wx, w2b0, w2b1, w2cp, w2d, b2.reshape(1, D),
        a2a, a2b, bA2.reshape(1, D))


# speedup vs baseline: 13.6591x; 1.9646x over previous
import jax
import jax.numpy as jnp
from jax import lax
from jax.experimental import pallas as pl
from jax.experimental.pallas import tpu as pltpu
from jax.experimental.pallas import tpu_sc as plsc

N = 10000
E = 320000
D = 128
NP = 10240
P = 3
C = 32

XS = 80
X1 = D - XS
WR = 80
GB = X1

NS = 16
K = 128
ER = 2512
NCH = ER // NS
RPT = NP // NS


def _sc_body(ei_hbm, ej_hbm, t0_hbm, t1_hbm, pp_hbm, z0_hbm, z1_hbm,
             s0_out, s1_out,
             ei3, ej3, rbuf, pbi, s0_sh, s1_sh, gsem, psem, isem, ssem):
    c = lax.axis_index("c")
    s = lax.axis_index("s")
    lanes = lax.iota(jnp.int32, 16)
    r0 = s * RPT

    @pl.when(c == 0)
    def _():
        pltpu.sync_copy(z0_hbm.at[pl.ds(r0, RPT)], s0_sh.at[pl.ds(r0, RPT)])

    @pl.when(c == 1)
    def _():
        pltpu.sync_copy(z1_hbm.at[pl.ds(r0, RPT)], s1_sh.at[pl.ds(r0, RPT)])

    plsc.subcore_barrier()

    def issue_idx(m, slot):
        r = s + 16 * m
        pltpu.async_copy(ei_hbm.at[pl.ds(r, 1)], ei3.at[pl.ds(slot, 1)], isem)
        pltpu.async_copy(ej_hbm.at[pl.ds(r, 1)], ej3.at[pl.ds(slot, 1)], isem)

    def wait_idx():
        pltpu.make_async_copy(
            ei_hbm.at[pl.ds(0, 1)], ei3.at[pl.ds(0, 1)], isem).wait()
        pltpu.make_async_copy(
            ej_hbm.at[pl.ds(0, 1)], ej3.at[pl.ds(0, 1)], isem).wait()

    def issue_gather(b, slot):
        @pl.when(c == 0)
        def _():
            pltpu.async_copy(t0_hbm.at[ej3.at[slot]], rbuf.at[b], gsem)

        @pl.when(c == 1)
        def _():
            pltpu.async_copy(t1_hbm.at[ej3.at[slot]], rbuf.at[b], gsem)
            pltpu.async_copy(pp_hbm.at[ei3.at[slot]], pbi.at[b], psem)

    def wait_gather():
        pltpu.make_async_copy(
            t0_hbm.at[ej3.at[0]], rbuf.at[0], gsem).wait()

        @pl.when(c == 1)
        def _():
            pltpu.make_async_copy(
                pp_hbm.at[ei3.at[0]], pbi.at[0], psem).wait()

    def issue_scatter(b, slot):
        @pl.when(c == 0)
        def _():
            pltpu.async_copy(rbuf.at[b], s0_sh.at[ei3.at[slot]], ssem,
                             add=True)

        @pl.when(c == 1)
        def _():
            pltpu.async_copy(rbuf.at[b], s1_sh.at[ei3.at[slot]], ssem,
                             add=True)

    def wait_scatter():
        pltpu.make_async_copy(rbuf.at[0], s0_sh.at[ei3.at[0]], ssem).wait()

    r = s
    pltpu.sync_copy(ei_hbm.at[pl.ds(r, 1)], ei3.at[pl.ds(0, 1)])
    pltpu.sync_copy(ej_hbm.at[pl.ds(r, 1)], ej3.at[pl.ds(0, 1)])
    issue_gather(0, 0)
    issue_idx(1, 1)

    def step(m, carry):
        b = lax.rem(m, 2)
        nb = 1 - b
        s_cur = lax.rem(m, 3)
        s_nxt = lax.rem(m + 1, 3)
        s_new = lax.rem(m + 2, 3)

        wait_gather()

        @pl.when(c == 1)
        def _():
            bvec = jnp.full((16,), 0, jnp.int32) + b
            for tt in range(K // 16):
                rows = tt * 16 + lanes
                col0 = jnp.zeros((16,), jnp.int32)
                pix = plsc.load_gather(pbi, [bvec, rows, col0])
                piy = plsc.load_gather(pbi, [bvec, rows, col0 + 1])
                piz = plsc.load_gather(pbi, [bvec, rows, col0 + 2])
                pjx = plsc.load_gather(rbuf, [bvec, rows, col0 + GB])
                pjy = plsc.load_gather(rbuf, [bvec, rows, col0 + GB + 1])
                pjz = plsc.load_gather(rbuf, [bvec, rows, col0 + GB + 2])
                dx = pjx - pix
                dy = pjy - piy
                dz = pjz - piz
                d2 = dx * dx + dy * dy + dz * dz
                yi = jnp.int32(0x5F3759DF) - lax.shift_right_logical(
                    plsc.bitcast(d2, jnp.int32), 1)
                y = plsc.bitcast(yi, jnp.float32)
                h = d2 * 0.5
                for _ in range(4):
                    y = y * (1.5 - h * y * y)
                one = jnp.full((16,), 1.0, jnp.float32)
                plsc.store_scatter(rbuf, [bvec, rows, col0 + GB], one)
                plsc.store_scatter(rbuf, [bvec, rows, col0 + GB + 1], d2 * y)
                plsc.store_scatter(rbuf, [bvec, rows, col0 + GB + 2], dx * y)
                plsc.store_scatter(rbuf, [bvec, rows, col0 + GB + 3], dy * y)
                plsc.store_scatter(rbuf, [bvec, rows, col0 + GB + 4], dz * y)

        issue_scatter(b, s_cur)

        @pl.when(m > 0)
        def _():
            wait_scatter()

        @pl.when(m < NCH - 2)
        def _():
            issue_idx(m + 2, s_new)

        @pl.when(m < NCH - 1)
        def _():
            wait_idx()
            issue_gather(nb, s_nxt)

        return carry

    lax.fori_loop(0, NCH, step, 0)

    wait_scatter()
    plsc.subcore_barrier()

    @pl.when(c == 0)
    def _():
        pltpu.sync_copy(s0_sh.at[pl.ds(r0, RPT)], s0_out.at[pl.ds(r0, RPT)])

    @pl.when(c == 1)
    def _():
        pltpu.sync_copy(s1_sh.at[pl.ds(r0, RPT)], s1_out.at[pl.ds(r0, RPT)])


def _sc_segment_sums(ei, ej, t0, t1, pp, z0, z1):
    mesh = plsc.VectorSubcoreMesh(core_axis_name="c", subcore_axis_name="s")
    return pl.kernel(
        _sc_body,
        out_type=(
            jax.ShapeDtypeStruct((NP, WR), jnp.float32),
            jax.ShapeDtypeStruct((NP, WR), jnp.float32),
        ),
        mesh=mesh,
        compiler_params=pltpu.CompilerParams(
            needs_layout_passes=False, use_tc_tiling_on_sc=False),
        scratch_types=[
            pltpu.VMEM((3, K), jnp.int32),
            pltpu.VMEM((3, K), jnp.int32),
            pltpu.VMEM((2, K, WR), jnp.float32),
            pltpu.VMEM((2, K, 16), jnp.float32),
            pltpu.VMEM_SHARED((NP, WR), jnp.float32),
            pltpu.VMEM_SHARED((NP, WR), jnp.float32),
            pltpu.SemaphoreType.DMA,
            pltpu.SemaphoreType.DMA,
            pltpu.SemaphoreType.DMA,
            pltpu.SemaphoreType.DMA,
        ],
    )(ei, ej, t0, t1, pp, z0, z1)


def _tc_body(s0_ref, s1_ref, x_ref, w1p_ref, b1_ref, a1_ref, ba1_ref,
             w2a_ref, w2b0_ref, w2b1_ref, w2cp_ref, w2d_ref, b2_ref,
             a2a_ref, a2b_ref, ba2_ref, out_ref):
    f32 = jnp.float32
    s0 = s0_ref[...]
    s1 = s1_ref[...]
    g = s1[:, GB:GB + 16]
    cnt = g[:, :1]
    inv = 1.0 / jnp.maximum(cnt, 1.0)
    has = jnp.minimum(cnt, 1.0)
    aggr1 = jnp.dot(g, w1p_ref[...], preferred_element_type=f32) * inv \
        + has * b1_ref[...]
    ctx = jnp.dot(aggr1, a1_ref[...], preferred_element_type=f32) + ba1_ref[...]
    t = jnp.dot(x_ref[...], w2a_ref[...], preferred_element_type=f32) \
        + jnp.dot(ctx, w2d_ref[...], preferred_element_type=f32) + b2_ref[...]
    sterm = jnp.dot(s0, w2b0_ref[...], preferred_element_type=f32) \
        + jnp.dot(s1[:, :X1], w2b1_ref[...], preferred_element_type=f32)
    aggr2 = (cnt * t + sterm
             + jnp.dot(g, w2cp_ref[...], preferred_element_type=f32)) * inv
    out_ref[...] = jnp.dot(aggr2, a2a_ref[...], preferred_element_type=f32) \
        + jnp.dot(ctx, a2b_ref[...], preferred_element_type=f32) + ba2_ref[...]


def _tc_combine(s0, s1, x, w1p, b1, a1, ba1, wx, w2b0, w2b1, w2cp, w2d, b2,
                a2a, a2b, ba2):
    bn = 1000
    grid = (N // bn,)
    full = lambda shape: pl.BlockSpec(shape, lambda i: (0,) * len(shape))
    return pl.pallas_call(
        _tc_body,
        grid=grid,
        in_specs=[
            pl.BlockSpec((bn, WR), lambda i: (i, 0)),
            pl.BlockSpec((bn, WR), lambda i: (i, 0)),
            pl.BlockSpec((bn, D), lambda i: (i, 0)),
            full((16, C)), full((1, C)), full((C, C)), full((1, C)),
            full((D, D)), full((XS, D)), full((X1, D)), full((16, D)),
            full((C, D)), full((1, D)), full((D, D)), full((C, D)),
            full((1, D)),
        ],
        out_specs=pl.BlockSpec((bn, D), lambda i: (i, 0)),
        out_shape=jax.ShapeDtypeStruct((N, D), jnp.float32),
    )(s0, s1, x, w1p, b1, a1, ba1, wx, w2b0, w2b1, w2cp, w2d, b2,
      a2a, a2b, ba2)


def kernel(x, edge_index, pos, W1, b1, A1, bA1, W2, b2, A2, bA2):
    ei = edge_index[0]
    ej = edge_index[1]
    npad = ER * K - E
    eip = jnp.concatenate([ei, jnp.full((npad,), NP - 1, jnp.int32)])
    ejp = jnp.concatenate([ej, jnp.zeros((npad,), jnp.int32)])
    eip = eip.reshape(ER, K)
    ejp = ejp.reshape(ER, K)
    t0 = jnp.zeros((NP, WR), jnp.float32).at[:N].set(x[:, :XS])
    t1 = jnp.zeros((NP, WR), jnp.float32).at[:N, :X1].set(x[:, XS:]) \
        .at[:N, GB:GB + P].set(pos)
    pp = jnp.zeros((NP, 16), jnp.float32).at[:N, :P].set(pos)
    z0 = jnp.zeros((NP, WR), jnp.float32)
    z1 = jnp.zeros((NP, WR), jnp.float32)

    s0, s1 = _sc_segment_sums(eip, ejp, t0, t1, pp, z0, z1)

    w1p = jnp.zeros((16, C), jnp.float32).at[1:1 + P + 1].set(W1)
    wx = W2[:D] - W2[D:2 * D]
    w2b0 = W2[D:D + XS]
    w2b1 = W2[D + XS:2 * D]
    w2cp = jnp.zeros((16, D), jnp.float32).at[1:1 + P + 1].set(
        W2[2 * D:2 * D + P + 1])
    w2d = W2[2 * D + P + 1:]
    a2a = A2[:D]
    a2b = A2[D:]

    return _tc_combine(
        s0, s1, x,
        w1p, b1.reshape(1, C), A1, bA1.reshape(1, C),
        wx, w2b0, w2b1, w2cp, w2d, b2.reshape(1, D),
        a2a, a2b, bA2.reshape(1, D))


# speedup vs baseline: 14.5539x; 1.0655x over previous
import jax
import jax.numpy as jnp
from jax import lax
from jax.experimental import pallas as pl
from jax.experimental.pallas import tpu as pltpu
from jax.experimental.pallas import tpu_sc as plsc

N = 10000
E = 320000
D = 128
NP = 10112
P = 3
C = 32

XS = 80
X1 = D - XS
W0 = 80
W1R = 64
GB = X1

NS = 16
K = 128
ER = 2512
NCH = ER // NS
RPT = NP // NS


def _sc_body(ei_hbm, ej_hbm, t0_hbm, t1_hbm, pp_hbm, z0_hbm, z1_hbm,
             s0_out, s1_out,
             ei3, ej3, rb0, rb1, pbi, s0_sh, s1_sh, gsem, psem, isem, ssem):
    c = lax.axis_index("c")
    s = lax.axis_index("s")
    lanes = lax.iota(jnp.int32, 16)
    r0 = s * RPT

    @pl.when(c == 0)
    def _():
        pltpu.sync_copy(z0_hbm.at[pl.ds(r0, RPT)], s0_sh.at[pl.ds(r0, RPT)])

    @pl.when(c == 1)
    def _():
        pltpu.sync_copy(z1_hbm.at[pl.ds(r0, RPT)], s1_sh.at[pl.ds(r0, RPT)])

    plsc.subcore_barrier()

    def issue_idx(m, slot):
        r = s + 16 * m
        pltpu.async_copy(ei_hbm.at[pl.ds(r, 1)], ei3.at[pl.ds(slot, 1)], isem)
        pltpu.async_copy(ej_hbm.at[pl.ds(r, 1)], ej3.at[pl.ds(slot, 1)], isem)

    def wait_idx():
        pltpu.make_async_copy(
            ei_hbm.at[pl.ds(0, 1)], ei3.at[pl.ds(0, 1)], isem).wait()
        pltpu.make_async_copy(
            ej_hbm.at[pl.ds(0, 1)], ej3.at[pl.ds(0, 1)], isem).wait()

    def issue_gather(b, slot):
        @pl.when(c == 0)
        def _():
            pltpu.async_copy(t0_hbm.at[ej3.at[slot]], rb0.at[b], gsem)

        @pl.when(c == 1)
        def _():
            pltpu.async_copy(t1_hbm.at[ej3.at[slot]], rb1.at[b], gsem)
            pltpu.async_copy(pp_hbm.at[ei3.at[slot]], pbi, psem)

    def wait_gather():
        @pl.when(c == 0)
        def _():
            pltpu.make_async_copy(
                t0_hbm.at[ej3.at[0]], rb0.at[0], gsem).wait()

        @pl.when(c == 1)
        def _():
            pltpu.make_async_copy(
                t1_hbm.at[ej3.at[0]], rb1.at[0], gsem).wait()
            pltpu.make_async_copy(
                pp_hbm.at[ei3.at[0]], pbi, psem).wait()

    def issue_scatter(b, slot):
        @pl.when(c == 0)
        def _():
            pltpu.async_copy(rb0.at[b], s0_sh.at[ei3.at[slot]], ssem,
                             add=True)

        @pl.when(c == 1)
        def _():
            pltpu.async_copy(rb1.at[b], s1_sh.at[ei3.at[slot]], ssem,
                             add=True)

    def wait_scatter():
        @pl.when(c == 0)
        def _():
            pltpu.make_async_copy(
                rb0.at[0], s0_sh.at[ei3.at[0]], ssem).wait()

        @pl.when(c == 1)
        def _():
            pltpu.make_async_copy(
                rb1.at[0], s1_sh.at[ei3.at[0]], ssem).wait()

    r = s
    pltpu.sync_copy(ei_hbm.at[pl.ds(r, 1)], ei3.at[pl.ds(0, 1)])
    pltpu.sync_copy(ej_hbm.at[pl.ds(r, 1)], ej3.at[pl.ds(0, 1)])
    issue_gather(0, 0)
    issue_idx(1, 1)

    def step(m, carry):
        b = lax.rem(m, 2)
        nb = 1 - b
        s_cur = lax.rem(m, 3)
        s_nxt = lax.rem(m + 1, 3)
        s_new = lax.rem(m + 2, 3)

        wait_gather()

        @pl.when(c == 1)
        def _():
            bvec = jnp.full((16,), 0, jnp.int32) + b
            for tt in range(K // 16):
                rows = tt * 16 + lanes
                col0 = jnp.zeros((16,), jnp.int32)
                pix = plsc.load_gather(pbi, [rows, col0])
                piy = plsc.load_gather(pbi, [rows, col0 + 1])
                piz = plsc.load_gather(pbi, [rows, col0 + 2])
                pjx = plsc.load_gather(rb1, [bvec, rows, col0 + GB])
                pjy = plsc.load_gather(rb1, [bvec, rows, col0 + GB + 1])
                pjz = plsc.load_gather(rb1, [bvec, rows, col0 + GB + 2])
                dx = pjx - pix
                dy = pjy - piy
                dz = pjz - piz
                d2 = dx * dx + dy * dy + dz * dz
                yi = jnp.int32(0x5F3759DF) - lax.shift_right_logical(
                    plsc.bitcast(d2, jnp.int32), 1)
                y = plsc.bitcast(yi, jnp.float32)
                h = d2 * 0.5
                for _ in range(3):
                    y = y * (1.5 - h * y * y)
                one = jnp.full((16,), 1.0, jnp.float32)
                plsc.store_scatter(rb1, [bvec, rows, col0 + GB], one)
                plsc.store_scatter(rb1, [bvec, rows, col0 + GB + 1], d2 * y)
                plsc.store_scatter(rb1, [bvec, rows, col0 + GB + 2], dx * y)
                plsc.store_scatter(rb1, [bvec, rows, col0 + GB + 3], dy * y)
                plsc.store_scatter(rb1, [bvec, rows, col0 + GB + 4], dz * y)

        issue_scatter(b, s_cur)

        @pl.when(m > 0)
        def _():
            wait_scatter()

        @pl.when(m < NCH - 2)
        def _():
            issue_idx(m + 2, s_new)

        @pl.when(m < NCH - 1)
        def _():
            wait_idx()
            issue_gather(nb, s_nxt)

        return carry

    lax.fori_loop(0, NCH, step, 0)

    wait_scatter()
    plsc.subcore_barrier()

    @pl.when(c == 0)
    def _():
        pltpu.sync_copy(s0_sh.at[pl.ds(r0, RPT)], s0_out.at[pl.ds(r0, RPT)])

    @pl.when(c == 1)
    def _():
        pltpu.sync_copy(s1_sh.at[pl.ds(r0, RPT)], s1_out.at[pl.ds(r0, RPT)])


def _sc_segment_sums(ei, ej, t0, t1, pp, z0, z1):
    mesh = plsc.VectorSubcoreMesh(core_axis_name="c", subcore_axis_name="s")
    return pl.kernel(
        _sc_body,
        out_type=(
            jax.ShapeDtypeStruct((NP, W0), jnp.float32),
            jax.ShapeDtypeStruct((NP, W1R), jnp.float32),
        ),
        mesh=mesh,
        compiler_params=pltpu.CompilerParams(
            needs_layout_passes=False, use_tc_tiling_on_sc=False),
        scratch_types=[
            pltpu.VMEM((3, K), jnp.int32),
            pltpu.VMEM((3, K), jnp.int32),
            pltpu.VMEM((2, K, W0), jnp.float32),
            pltpu.VMEM((2, K, W1R), jnp.float32),
            pltpu.VMEM((K, 16), jnp.float32),
            pltpu.VMEM_SHARED((NP, W0), jnp.float32),
            pltpu.VMEM_SHARED((NP, W1R), jnp.float32),
            pltpu.SemaphoreType.DMA,
            pltpu.SemaphoreType.DMA,
            pltpu.SemaphoreType.DMA,
            pltpu.SemaphoreType.DMA,
        ],
    )(ei, ej, t0, t1, pp, z0, z1)


def _tc_body(s0_ref, s1_ref, x_ref, w1p_ref, b1_ref, a1_ref, ba1_ref,
             w2a_ref, w2b0_ref, w2b1_ref, w2cp_ref, w2d_ref, b2_ref,
             a2a_ref, a2b_ref, ba2_ref, out_ref):
    f32 = jnp.float32
    s0 = s0_ref[...]
    s1 = s1_ref[...]
    g = s1[:, GB:GB + 16]
    cnt = g[:, :1]
    inv = 1.0 / jnp.maximum(cnt, 1.0)
    has = jnp.minimum(cnt, 1.0)
    aggr1 = jnp.dot(g, w1p_ref[...], preferred_element_type=f32) * inv \
        + has * b1_ref[...]
    ctx = jnp.dot(aggr1, a1_ref[...], preferred_element_type=f32) + ba1_ref[...]
    t = jnp.dot(x_ref[...], w2a_ref[...], preferred_element_type=f32) \
        + jnp.dot(ctx, w2d_ref[...], preferred_element_type=f32) + b2_ref[...]
    sterm = jnp.dot(s0, w2b0_ref[...], preferred_element_type=f32) \
        + jnp.dot(s1[:, :X1], w2b1_ref[...], preferred_element_type=f32)
    aggr2 = (cnt * t + sterm
             + jnp.dot(g, w2cp_ref[...], preferred_element_type=f32)) * inv
    out_ref[...] = jnp.dot(aggr2, a2a_ref[...], preferred_element_type=f32) \
        + jnp.dot(ctx, a2b_ref[...], preferred_element_type=f32) + ba2_ref[...]


def _tc_combine(s0, s1, x, w1p, b1, a1, ba1, wx, w2b0, w2b1, w2cp, w2d, b2,
                a2a, a2b, ba2):
    bn = 1000
    grid = (N // bn,)
    full = lambda shape: pl.BlockSpec(shape, lambda i: (0,) * len(shape))
    return pl.pallas_call(
        _tc_body,
        grid=grid,
        in_specs=[
            pl.BlockSpec((bn, W0), lambda i: (i, 0)),
            pl.BlockSpec((bn, W1R), lambda i: (i, 0)),
            pl.BlockSpec((bn, D), lambda i: (i, 0)),
            full((16, C)), full((1, C)), full((C, C)), full((1, C)),
            full((D, D)), full((XS, D)), full((X1, D)), full((16, D)),
            full((C, D)), full((1, D)), full((D, D)), full((C, D)),
            full((1, D)),
        ],
        out_specs=pl.BlockSpec((bn, D), lambda i: (i, 0)),
        out_shape=jax.ShapeDtypeStruct((N, D), jnp.float32),
    )(s0, s1, x, w1p, b1, a1, ba1, wx, w2b0, w2b1, w2cp, w2d, b2,
      a2a, a2b, ba2)


def kernel(x, edge_index, pos, W1, b1, A1, bA1, W2, b2, A2, bA2):
    ei = edge_index[0]
    ej = edge_index[1]
    npad = ER * K - E
    eip = jnp.concatenate([ei, jnp.full((npad,), NP - 1, jnp.int32)])
    ejp = jnp.concatenate([ej, jnp.zeros((npad,), jnp.int32)])
    eip = eip.reshape(ER, K)
    ejp = ejp.reshape(ER, K)
    t0 = x[:, :XS]
    t1 = jnp.concatenate(
        [x[:, XS:], pos, jnp.zeros((N, W1R - X1 - P), jnp.float32)], axis=1)
    pp = jnp.zeros((NP, 16), jnp.float32).at[:N, :P].set(pos)
    z0 = jnp.zeros((NP, W0), jnp.float32)
    z1 = jnp.zeros((NP, W1R), jnp.float32)

    s0, s1 = _sc_segment_sums(eip, ejp, t0, t1, pp, z0, z1)

    w1p = jnp.zeros((16, C), jnp.float32).at[1:1 + P + 1].set(W1)
    wx = W2[:D] - W2[D:2 * D]
    w2b0 = W2[D:D + XS]
    w2b1 = W2[D + XS:2 * D]
    w2cp = jnp.zeros((16, D), jnp.float32).at[1:1 + P + 1].set(
        W2[2 * D:2 * D + P + 1])
    w2d = W2[2 * D + P + 1:]
    a2a = A2[:D]
    a2b = A2[D:]

    return _tc_combine(
        s0, s1, x,
        w1p, b1.reshape(1, C), A1, bA1.reshape(1, C),
        wx, w2b0, w2b1, w2cp, w2d, b2.reshape(1, D),
        a2a, a2b, bA2.reshape(1, D))
